# Initial kernel scaffold; baseline (speedup 1.0000x reference)
import jax
import jax.numpy as jnp
from jax.experimental import pallas as pl


def kernel(seq1, seq2, adj, W_gcn, b_gcn, W_disc, b_disc):
    raise NotImplementedError("write your pallas kernel here")


# speedup vs baseline: 2.0410x; 2.0410x over previous
import functools

import jax
import jax.numpy as jnp
from jax import lax
from jax.experimental import pallas as pl
from jax.experimental.pallas import tpu as pltpu
from jax.experimental.pallas import tpu_sc as plsc

N = 10000
D = 128
E = 320000
NC = 2
NS = 16
CH = 128
ROWS_PER_TILE = 160
ROWS = ROWS_PER_TILE * NS
E_PAD = ROWS * CH
NODES_PER_SC = 5120
ACC_ROWS = 5248
DUMMY = NODES_PER_SC
OUT_ROWS = 2 * NODES_PER_SC
ZERO_PER_TILE = ACC_ROWS // NS
ZB = 64
OUT_PER_TILE = NODES_PER_SC // NS

_BM = 1000


def _pre_body(x1_ref, x2_ref, w_ref, b_ref, o1_ref, o2_ref):
    w = w_ref[...]
    b = b_ref[...]
    o1_ref[...] = jnp.dot(x1_ref[...], w, preferred_element_type=jnp.float32) + b
    o2_ref[...] = jnp.dot(x2_ref[...], w, preferred_element_type=jnp.float32) + b


_pre_call = pl.pallas_call(
    _pre_body,
    grid=(N // _BM,),
    in_specs=[
        pl.BlockSpec((_BM, D), lambda i: (i, 0)),
        pl.BlockSpec((_BM, D), lambda i: (i, 0)),
        pl.BlockSpec((D, D), lambda i: (0, 0)),
        pl.BlockSpec((1, D), lambda i: (0, 0)),
    ],
    out_specs=[pl.BlockSpec((_BM, D), lambda i: (i, 0))] * 2,
    out_shape=[jax.ShapeDtypeStruct((N, D), jnp.float32)] * 2,
)


def _sc_body(pre1_h, pre2_h, src_h, dst_h, out1_h, out2_h,
             src_v, dst_v, gbuf, zbuf, acc, sem0, sem1):
    cid = lax.axis_index("c")
    sid = lax.axis_index("s")
    lo = cid * NODES_PER_SC

    def _zero(i, carry):
        r = i // (D // 16)
        c = i % (D // 16)
        zbuf[r, pl.ds(c * 16, 16)] = jnp.zeros((16,), jnp.float32)
        return carry

    lax.fori_loop(0, ZB * (D // 16), _zero, None)

    pltpu.sync_copy(src_h.at[pl.ds(sid * ROWS_PER_TILE, ROWS_PER_TILE)], src_v)
    pltpu.sync_copy(dst_h.at[pl.ds(sid * ROWS_PER_TILE, ROWS_PER_TILE)], dst_v)

    def _remap(i, carry):
        r = i // (CH // 16)
        c = i % (CH // 16)
        d = dst_v[r, pl.ds(c * 16, 16)]
        m = (d >= lo) & (d < lo + NODES_PER_SC)
        dst_v[r, pl.ds(c * 16, 16)] = jnp.where(m, d - lo, DUMMY)
        return carry

    lax.fori_loop(0, ROWS_PER_TILE * (CH // 16), _remap, None)

    sems = (sem0, sem1)

    def _process(pre_h):
        pltpu.async_copy(pre_h.at[src_v.at[0]], gbuf.at[0], sem0)
        pltpu.async_copy(pre_h.at[src_v.at[1]], gbuf.at[1], sem1)

        def _loop(j, carry):
            r0 = j * 2
            for b in range(2):
                r = r0 + b
                pltpu.make_async_copy(pre_h.at[src_v.at[r]], gbuf.at[b],
                                      sems[b]).wait()
                pltpu.sync_copy(gbuf.at[b], acc.at[dst_v.at[r]], add=True)
                pltpu.async_copy(pre_h.at[src_v.at[r + 2]], gbuf.at[b], sems[b])
            return carry

        lax.fori_loop(0, (ROWS_PER_TILE - 2) // 2, _loop, None)
        for b in range(2):
            r = ROWS_PER_TILE - 2 + b
            pltpu.make_async_copy(pre_h.at[src_v.at[r]], gbuf.at[b],
                                  sems[b]).wait()
            pltpu.sync_copy(gbuf.at[b], acc.at[dst_v.at[r]], add=True)

    pres = (pre1_h, pre2_h)
    outs = (out1_h, out2_h)
    zrow = sid * ZERO_PER_TILE
    row0 = sid * OUT_PER_TILE
    for phase in range(2):
        for k in range(ZERO_PER_TILE // ZB):
            pltpu.sync_copy(zbuf, acc.at[pl.ds(zrow + k * ZB, ZB)])
        rem = ZERO_PER_TILE % ZB
        if rem:
            pltpu.sync_copy(zbuf.at[pl.ds(0, rem)],
                            acc.at[pl.ds(zrow + ZERO_PER_TILE - rem, rem)])
        plsc.subcore_barrier()

        _process(pres[phase])

        plsc.subcore_barrier()

        pltpu.sync_copy(acc.at[pl.ds(row0, OUT_PER_TILE)],
                        outs[phase].at[pl.ds(lo + row0, OUT_PER_TILE)])

        plsc.subcore_barrier()


@functools.cache
def _sc_call():
    return pl.kernel(
        _sc_body,
        out_type=[jax.ShapeDtypeStruct((OUT_ROWS, D), jnp.float32)] * 2,
        mesh=plsc.VectorSubcoreMesh(core_axis_name="c", subcore_axis_name="s"),
        scratch_types=[
            pltpu.VMEM((ROWS_PER_TILE, CH), jnp.int32),
            pltpu.VMEM((ROWS_PER_TILE, CH), jnp.int32),
            pltpu.VMEM((2, CH, D), jnp.float32),
            pltpu.VMEM((ZB, D), jnp.float32),
            pltpu.VMEM_SHARED((ACC_ROWS, D), jnp.float32),
            pltpu.SemaphoreType.DMA,
            pltpu.SemaphoreType.DMA,
        ],
    )


def _disc_body(a1_ref, a2_ref, wd_ref, bd_ref, o1_ref, o2_ref):
    valid = lax.broadcasted_iota(jnp.int32, (OUT_ROWS, 1), 0) < N
    h1 = a1_ref[...]
    h1 = jnp.where(valid, jnp.where(h1 > 0, h1, 0.25 * h1), 0.0)
    h2 = a2_ref[...]
    h2 = jnp.where(h2 > 0, h2, 0.25 * h2)
    c = jax.nn.sigmoid(jnp.sum(h1, axis=0, keepdims=True) / N)
    wc = lax.dot_general(c, wd_ref[...], (((1,), (1,)), ((), ())))
    b = bd_ref[0, 0]
    o1_ref[...] = jnp.sum(h1 * wc, axis=1, keepdims=True) + b
    o2_ref[...] = jnp.sum(h2 * wc, axis=1, keepdims=True) + b


_disc_call = pl.pallas_call(
    _disc_body,
    out_shape=[jax.ShapeDtypeStruct((OUT_ROWS, 1), jnp.float32)] * 2,
)


def kernel(seq1, seq2, adj, W_gcn, b_gcn, W_disc, b_disc):
    pre1, pre2 = _pre_call(seq1, seq2, W_gcn, b_gcn.reshape(1, D))
    pad = E_PAD - E
    src = jnp.concatenate([adj[0], jnp.zeros((pad,), jnp.int32)]).reshape(ROWS, CH)
    dst = jnp.concatenate([adj[1], jnp.full((pad,), N, jnp.int32)]).reshape(ROWS, CH)
    agg1, agg2 = _sc_call()(pre1, pre2, src, dst)
    o1, o2 = _disc_call(agg1, agg2, W_disc, b_disc.reshape(1, 1))
    return jnp.concatenate([o1[:N, 0], o2[:N, 0]])


# speedup vs baseline: 2.1485x; 1.0527x over previous
import functools

import jax
import jax.numpy as jnp
from jax import lax
from jax.experimental import pallas as pl
from jax.experimental.pallas import tpu as pltpu
from jax.experimental.pallas import tpu_sc as plsc

N = 10000
D = 128
E = 320000
NC = 2
NS = 16
CH = 128
ROWS_PER_TILE = 160
ROWS = ROWS_PER_TILE * NS
E_PAD = ROWS * CH
NODES_PER_SC = 5120
ACC_ROWS = 5248
DUMMY = NODES_PER_SC
OUT_ROWS = 2 * NODES_PER_SC
ZERO_PER_TILE = ACC_ROWS // NS
ZB = 64
OUT_PER_TILE = NODES_PER_SC // NS

_BM = 1000


def _pre_body(x1_ref, x2_ref, w_ref, b_ref, o1_ref, o2_ref):
    w = w_ref[...]
    b = b_ref[...]
    o1_ref[...] = jnp.dot(x1_ref[...], w, preferred_element_type=jnp.float32) + b
    o2_ref[...] = jnp.dot(x2_ref[...], w, preferred_element_type=jnp.float32) + b


_pre_call = pl.pallas_call(
    _pre_body,
    grid=(N // _BM,),
    in_specs=[
        pl.BlockSpec((_BM, D), lambda i: (i, 0)),
        pl.BlockSpec((_BM, D), lambda i: (i, 0)),
        pl.BlockSpec((D, D), lambda i: (0, 0)),
        pl.BlockSpec((1, D), lambda i: (0, 0)),
    ],
    out_specs=[pl.BlockSpec((_BM, D), lambda i: (i, 0))] * 2,
    out_shape=[jax.ShapeDtypeStruct((N, D), jnp.float32)] * 2,
)


def _sc_body(pre1_h, pre2_h, src_h, dst_h, out1_h, out2_h,
             src_v, dst_v, gbuf, zbuf, acc, sem0, sem1):
    cid = lax.axis_index("c")
    sid = lax.axis_index("s")
    lo = cid * NODES_PER_SC

    def _zero(i, carry):
        r = i // (D // 16)
        c = i % (D // 16)
        zbuf[r, pl.ds(c * 16, 16)] = jnp.zeros((16,), jnp.float32)
        return carry

    lax.fori_loop(0, ZB * (D // 16), _zero, None)

    pltpu.sync_copy(src_h.at[pl.ds(sid * ROWS_PER_TILE, ROWS_PER_TILE)], src_v)
    pltpu.sync_copy(dst_h.at[pl.ds(sid * ROWS_PER_TILE, ROWS_PER_TILE)], dst_v)

    def _remap(i, carry):
        r = i // (CH // 16)
        c = i % (CH // 16)
        d = dst_v[r, pl.ds(c * 16, 16)]
        s = src_v[r, pl.ds(c * 16, 16)]
        m = (d >= lo) & (d < lo + NODES_PER_SC)
        dst_v[r, pl.ds(c * 16, 16)] = jnp.where(m, d - lo, DUMMY + (s & 127))
        return carry

    lax.fori_loop(0, ROWS_PER_TILE * (CH // 16), _remap, None)

    sems = (sem0, sem1)

    def _process(pre_h):
        pltpu.async_copy(pre_h.at[src_v.at[0]], gbuf.at[0], sem0)
        pltpu.async_copy(pre_h.at[src_v.at[1]], gbuf.at[1], sem1)

        def _loop(j, carry):
            r0 = j * 2
            for b in range(2):
                r = r0 + b
                pltpu.make_async_copy(pre_h.at[src_v.at[r]], gbuf.at[b],
                                      sems[b]).wait()
                pltpu.sync_copy(gbuf.at[b], acc.at[dst_v.at[r]], add=True)
                pltpu.async_copy(pre_h.at[src_v.at[r + 2]], gbuf.at[b], sems[b])
            return carry

        lax.fori_loop(0, (ROWS_PER_TILE - 2) // 2, _loop, None)
        for b in range(2):
            r = ROWS_PER_TILE - 2 + b
            pltpu.make_async_copy(pre_h.at[src_v.at[r]], gbuf.at[b],
                                  sems[b]).wait()
            pltpu.sync_copy(gbuf.at[b], acc.at[dst_v.at[r]], add=True)

    pres = (pre1_h, pre2_h)
    outs = (out1_h, out2_h)
    zrow = sid * ZERO_PER_TILE
    row0 = sid * OUT_PER_TILE
    for phase in range(2):
        for k in range(ZERO_PER_TILE // ZB):
            pltpu.sync_copy(zbuf, acc.at[pl.ds(zrow + k * ZB, ZB)])
        rem = ZERO_PER_TILE % ZB
        if rem:
            pltpu.sync_copy(zbuf.at[pl.ds(0, rem)],
                            acc.at[pl.ds(zrow + ZERO_PER_TILE - rem, rem)])
        plsc.subcore_barrier()

        _process(pres[phase])

        plsc.subcore_barrier()

        pltpu.sync_copy(acc.at[pl.ds(row0, OUT_PER_TILE)],
                        outs[phase].at[pl.ds(lo + row0, OUT_PER_TILE)])

        plsc.subcore_barrier()


@functools.cache
def _sc_call():
    return pl.kernel(
        _sc_body,
        out_type=[jax.ShapeDtypeStruct((OUT_ROWS, D), jnp.float32)] * 2,
        mesh=plsc.VectorSubcoreMesh(core_axis_name="c", subcore_axis_name="s"),
        scratch_types=[
            pltpu.VMEM((ROWS_PER_TILE, CH), jnp.int32),
            pltpu.VMEM((ROWS_PER_TILE, CH), jnp.int32),
            pltpu.VMEM((2, CH, D), jnp.float32),
            pltpu.VMEM((ZB, D), jnp.float32),
            pltpu.VMEM_SHARED((ACC_ROWS, D), jnp.float32),
            pltpu.SemaphoreType.DMA,
            pltpu.SemaphoreType.DMA,
        ],
    )


def _disc_body(a1_ref, a2_ref, wd_ref, bd_ref, o1_ref, o2_ref):
    valid = lax.broadcasted_iota(jnp.int32, (OUT_ROWS, 1), 0) < N
    h1 = a1_ref[...]
    h1 = jnp.where(valid, jnp.where(h1 > 0, h1, 0.25 * h1), 0.0)
    h2 = a2_ref[...]
    h2 = jnp.where(h2 > 0, h2, 0.25 * h2)
    c = jax.nn.sigmoid(jnp.sum(h1, axis=0, keepdims=True) / N)
    wc = lax.dot_general(c, wd_ref[...], (((1,), (1,)), ((), ())))
    b = bd_ref[0, 0]
    o1_ref[...] = jnp.sum(h1 * wc, axis=1, keepdims=True) + b
    o2_ref[...] = jnp.sum(h2 * wc, axis=1, keepdims=True) + b


_disc_call = pl.pallas_call(
    _disc_body,
    out_shape=[jax.ShapeDtypeStruct((OUT_ROWS, 1), jnp.float32)] * 2,
)


def kernel(seq1, seq2, adj, W_gcn, b_gcn, W_disc, b_disc):
    pre1, pre2 = _pre_call(seq1, seq2, W_gcn, b_gcn.reshape(1, D))
    pad = E_PAD - E
    src = jnp.concatenate([adj[0], jnp.zeros((pad,), jnp.int32)]).reshape(ROWS, CH)
    dst = jnp.concatenate([adj[1], jnp.full((pad,), N, jnp.int32)]).reshape(ROWS, CH)
    agg1, agg2 = _sc_call()(pre1, pre2, src, dst)
    o1, o2 = _disc_call(agg1, agg2, W_disc, b_disc.reshape(1, 1))
    return jnp.concatenate([o1[:N, 0], o2[:N, 0]])
